# baseline (speedup 1.0000x reference)
import functools

import jax
import jax.numpy as jnp
import numpy as np
from jax import lax
from jax.experimental import pallas as pl
from jax.experimental.pallas import tpu as pltpu
from jax.experimental.pallas import tpu_sc as plsc

B, L, D = 64, 512, 1024
NW = 32
BPW = B // NW
RPW = BPW * L
K = 16
NCHUNK = RPW // K
NLANE = 16

_i32 = jnp.int32
_f32 = jnp.float32


def _consts():
    rng = np.random.RandomState(0)
    c0 = np.zeros(B, np.int32)
    c1 = np.zeros(B, np.int32)
    s = np.zeros(B, np.float32)
    for b in range(B):
        cdt = rng.choice(B, 2, replace=False)
        c0[b], c1[b] = cdt[0], cdt[1]
        s[b] = float(rng.uniform(0, 1, 1)[0])
    r = np.array([int(rng.choice([0, 1, 2])) for _ in range(B)], np.int32)
    return c0, c1, s, r


def _body(ids, attn, zf, label, nlab, c0, c1, rr, ss,
          oz, oattn, olabel, onlab,
          blk_v, pads_v, ast_v, len_v, chg_v,
          c0_v, c1_v, r_v, s_v, nlab_v, label_v,
          attn2_v, attnout_v, idx1_v, idx2_v, w1_v,
          g1, g2, onlab_v, red_v, *allsems):
    wid = lax.axis_index("s") * 2 + lax.axis_index("c")
    iota = lax.iota(_i32, NLANE)
    lane0 = iota == 0

    def full(x, dtype=_i32):
        return jnp.full((NLANE,), x, dtype=dtype)

    pltpu.sync_copy(c0, c0_v)
    pltpu.sync_copy(c1, c1_v)
    pltpu.sync_copy(rr, r_v)
    pltpu.sync_copy(ss, s_v)
    pltpu.sync_copy(nlab, nlab_v)

    def scan_first_zero(src, dst_v):
        nblk = B // 8
        pltpu.async_copy(src.at[pl.ds(0, 8)], blk_v.at[0], allsems[0])
        for blk in range(nblk):
            slot = blk % 2
            pltpu.make_async_copy(src.at[pl.ds(0, 8)], blk_v.at[slot],
                                  allsems[slot]).wait()
            if blk + 1 < nblk:
                pltpu.async_copy(src.at[pl.ds((blk + 1) * 8, 8)],
                                 blk_v.at[1 - slot], allsems[1 - slot])
            for t in range(8):
                def cond(st):
                    j, f = st
                    return (f == L) & (j < L // NLANE)

                def bdy(st):
                    j, f = st
                    v = blk_v[slot, t, pl.ds(j * NLANE, NLANE)]
                    g = jnp.min(jnp.where(v == 0, iota + j * NLANE, L))
                    return (j + 1, g)

                _, f = lax.while_loop(cond, bdy, (0, L))
                first = jnp.where(f == L, 0, f)
                plsc.store_scatter(dst_v, [full(blk * 8 + t)], full(first),
                                   mask=lane0)

    scan_first_zero(ids, pads_v)
    ast_v = pads_v

    for g in range(B // NLANE):
        sl = pl.ds(g * NLANE, NLANE)
        p0 = plsc.load_gather(pads_v, [c0_v[sl]])
        p1 = plsc.load_gather(pads_v, [c1_v[sl]])
        len_v[sl] = (p0 + p1) >> 1
        chg_v[sl] = jnp.where((r_v[sl] != 0) & (nlab_v[sl] != 3), 1, 0)

    pltpu.sync_copy(attn.at[pl.ds(BPW * wid, BPW)], attn2_v)
    pe_s, le_s = [], []
    for k in range(BPW):
        b = BPW * wid + k
        bv = full(b)
        padb = plsc.load_gather(pads_v, [bv])
        astb = plsc.load_gather(ast_v, [bv])
        lenb = plsc.load_gather(len_v, [bv])
        chgb = plsc.load_gather(chg_v, [bv])
        rb = plsc.load_gather(r_v, [bv])
        c0b = plsc.load_gather(c0_v, [bv])
        c1b = plsc.load_gather(c1_v, [bv])
        sb = plsc.load_gather(s_v, [bv])
        pe = jnp.where(rb == 1, 0, padb)
        le = jnp.where(chgb == 1, lenb, 0)
        pe_s.append(pe[0])
        le_s.append(le[0])

        @pl.loop(0, L // NLANE)
        def _(j):
            pos = iota + j * NLANE
            inc = (pos >= pe) & (pos < pe + le)
            base = jnp.where(pos < pe, pos, jnp.maximum(pos - le, 0))
            cj = pos - pe
            i1 = jnp.where(inc, c0b * L + cj, bv * L + base)
            i2 = jnp.where(inc, c1b * L + cj, i1)
            w1 = jnp.where(inc, sb, 1.0)
            off = k * L + j * NLANE
            idx1_v[pl.ds(off, NLANE)] = i1
            idx2_v[pl.ds(off, NLANE)] = i2
            w1_v[pl.ds(off, NLANE)] = w1
            av = attn2_v[k, pl.ds(j * NLANE, NLANE)]
            attnout_v[k, pl.ds(j * NLANE, NLANE)] = jnp.where(
                (pos >= astb) & (pos < astb + le), 1, av)

    pltpu.sync_copy(attnout_v, oattn.at[pl.ds(BPW * wid, BPW)])

    @pl.when(wid == 0)
    def _():
        pltpu.sync_copy(label, label_v)
        for g in range(B // NLANE):
            sl = pl.ds(g * NLANE, NLANE)
            chg_g = chg_v[sl]
            onlab_v[sl] = nlab_v[sl] + chg_g
            plsc.store_scatter(label_v,
                               [iota + g * NLANE, full(label.shape[1] - 1)],
                               full(1), mask=chg_g == 1)
        pltpu.sync_copy(label_v, olabel)
        pltpu.sync_copy(onlab_v, onlab)

    base_row = wid * RPW
    gsems = allsems[:4]
    osems = allsems[4:]

    def conv_flag(c):
        in_b1 = c >= (NCHUNK // BPW)
        pe_c = jnp.where(in_b1, pe_s[1], pe_s[0])
        le_c = jnp.where(in_b1, le_s[1], le_s[0])
        ps = (c % (NCHUNK // BPW)) * K
        return (le_c > 0) & (pe_c < ps + K) & (ps < pe_c + le_c)

    def fire1(c, t):
        iv1 = idx1_v[pl.ds(c * K, K)]
        pltpu.async_copy(zf.at[iv1], g1.at[t], gsems[t])

    def fire2(c, t, t2):
        @pl.when((c < NCHUNK) & conv_flag(c))
        def _():
            iv2 = idx2_v[pl.ds(c * K, K)]
            pltpu.async_copy(zf.at[iv2], g2.at[t2], gsems[t])

    def drain(c, t, t2):
        pltpu.make_async_copy(zf.at[pl.ds(0, K)], g1.at[t], gsems[t]).wait()

        @pl.when(conv_flag(c))
        def _():
            pltpu.make_async_copy(zf.at[pl.ds(0, K)], g2.at[t2],
                                  gsems[t]).wait()

    def comp_store(c, t, t2):
        g1s = g1.at[t]
        g2s = g2.at[t2]

        @pl.when(conv_flag(c))
        def _():
            w1s = [plsc.load_gather(w1_v, [full(c * K + row)]) for row in range(K)]

            @pl.loop(0, D // NLANE)
            def _(j):
                sl = pl.ds(j * NLANE, NLANE)
                for row in range(K):
                    a = g1s[row, sl]
                    g1s[row, sl] = w1s[row] * a + (1.0 - w1s[row]) * g2s[row, sl]

        pltpu.async_copy(g1s, oz.at[pl.ds(base_row + c * K, K)], osems[t])

    for t in range(4):
        fire1(t, t)
    fire2(0, 0, 0)
    fire2(1, 1, 1)

    @pl.loop(0, NCHUNK, step=4)
    def _(c):
        for t in range(4):
            ct = c + t
            drain(ct, t, t % 2)
            comp_store(ct, t, t % 2)
            fire2(ct + 2, (t + 2) % 4, t % 2)
            tp = (t + 3) % 4

            @pl.when((ct + 3 >= 4) & (ct + 3 < NCHUNK))
            def _():
                pltpu.make_async_copy(g1.at[tp], oz.at[pl.ds(base_row, K)],
                                      osems[tp]).wait()
                fire1(ct + 3, tp)

    for t in range(4):
        pltpu.make_async_copy(g1.at[t], oz.at[pl.ds(base_row, K)],
                              osems[t]).wait()


@jax.jit
def kernel(input_ids, attn_mask, z, label_ids, n_label_ids):
    c0, c1, s, r = _consts()
    nlab_cols = label_ids.shape[1]

    kfn = pl.kernel(
        functools.partial(_body),
        out_type=[
            jax.ShapeDtypeStruct((B * L, D), _f32),
            jax.ShapeDtypeStruct((B, L), _i32),
            jax.ShapeDtypeStruct((B, nlab_cols), _i32),
            jax.ShapeDtypeStruct((B,), _i32),
        ],
        mesh=plsc.VectorSubcoreMesh(core_axis_name="c", subcore_axis_name="s"),
        compiler_params=pltpu.CompilerParams(needs_layout_passes=False),
        scratch_types=[
            pltpu.VMEM((2, 8, L), _i32),
            pltpu.VMEM((B,), _i32),
            pltpu.VMEM((B,), _i32),
            pltpu.VMEM((B,), _i32),
            pltpu.VMEM((B,), _i32),
            pltpu.VMEM((B,), _i32),
            pltpu.VMEM((B,), _i32),
            pltpu.VMEM((B,), _i32),
            pltpu.VMEM((B,), _f32),
            pltpu.VMEM((B,), _i32),
            pltpu.VMEM((B, nlab_cols), _i32),
            pltpu.VMEM((BPW, L), _i32),
            pltpu.VMEM((BPW, L), _i32),
            pltpu.VMEM((RPW,), _i32),
            pltpu.VMEM((RPW,), _i32),
            pltpu.VMEM((RPW,), _f32),
            pltpu.VMEM((4, K, D), _f32),
            pltpu.VMEM((2, K, D), _f32),
            pltpu.VMEM((B,), _i32),
            pltpu.VMEM((NLANE,), _i32),
            pltpu.SemaphoreType.DMA,
            pltpu.SemaphoreType.DMA,
            pltpu.SemaphoreType.DMA,
            pltpu.SemaphoreType.DMA,
            pltpu.SemaphoreType.DMA,
            pltpu.SemaphoreType.DMA,
            pltpu.SemaphoreType.DMA,
            pltpu.SemaphoreType.DMA,
        ],
    )
    zf = z.reshape(B * L, D)
    oz, oattn, olabel, onlab = kfn(
        input_ids, attn_mask, zf, label_ids, n_label_ids,
        jnp.asarray(c0), jnp.asarray(c1), jnp.asarray(r), jnp.asarray(s))
    return (oz.reshape(B, L, D), oattn.astype(attn_mask.dtype),
            olabel.astype(label_ids.dtype), onlab.astype(n_label_ids.dtype))
